# baseline (speedup 1.0000x reference)
import functools

import jax
import jax.numpy as jnp
from jax import lax
from jax.experimental import pallas as pl
from jax.experimental.pallas import tpu as pltpu
from jax.experimental.pallas import tpu_sc as plsc

N = 10000
E = 320000
D = 128

NC = 2
NS = 16
L = 16
NW = NC * NS
FPT = D // NW
EPT = 10240
EPAD = NW * EPT
NDEN = 10240
CH = 1280
NCHUNK = E // CH

CBLK = 1024
CGRID = -(-N // CBLK)

_f32 = jnp.float32


def _enc_body(xt_ref, wn_ref, bn_ref, w_ref, asr_ref, adr_ref,
              hwt_ref, as_ref, ad_ref):
    h = jnp.dot(wn_ref[...], xt_ref[...], preferred_element_type=_f32) + bn_ref[...]
    hw = jnp.dot(w_ref[...], h, preferred_element_type=_f32)
    hwt_ref[...] = hw
    as_ref[...] = jnp.dot(asr_ref[...], hw, preferred_element_type=_f32)
    ad_ref[...] = jnp.dot(adr_ref[...], hw, preferred_element_type=_f32)


def _encode(xt, wn, bn, w, asr, adr):
    return pl.pallas_call(
        _enc_body,
        grid=(CGRID,),
        in_specs=[
            pl.BlockSpec((D, CBLK), lambda i: (0, i)),
            pl.BlockSpec((D, D), lambda i: (0, 0)),
            pl.BlockSpec((D, 1), lambda i: (0, 0)),
            pl.BlockSpec((D, D), lambda i: (0, 0)),
            pl.BlockSpec((1, D), lambda i: (0, 0)),
            pl.BlockSpec((1, D), lambda i: (0, 0)),
        ],
        out_specs=[
            pl.BlockSpec((D, CBLK), lambda i: (0, i)),
            pl.BlockSpec((1, CBLK), lambda i: (0, i)),
            pl.BlockSpec((1, CBLK), lambda i: (0, i)),
        ],
        out_shape=[
            jax.ShapeDtypeStruct((D, N), _f32),
            jax.ShapeDtypeStruct((1, N), _f32),
            jax.ShapeDtypeStruct((1, N), _f32),
        ],
    )(xt, wn, bn, w, asr, adr)


def _fold(acc_ref, den_ref, hwt_ref, as_ref, ad_ref, b_ref):
    t = as_ref[...] + ad_ref[...]
    e = jnp.where(t >= 0, t, t * _f32(0.2))
    ws = jnp.exp(e)
    den = jnp.sum(den_ref[...], axis=0, keepdims=True)
    return (acc_ref[...] + ws * hwt_ref[...]) / (den + ws) + b_ref[...]


def _fold_enc_body(acc_ref, den_ref, hwt_ref, as_ref, ad_ref, b_ref,
                   w_ref, asr_ref, adr_ref, hwt2_ref, as2_ref, ad2_ref):
    h = _fold(acc_ref, den_ref, hwt_ref, as_ref, ad_ref, b_ref)
    hw = jnp.dot(w_ref[...], h, preferred_element_type=_f32)
    hwt2_ref[...] = hw
    as2_ref[...] = jnp.dot(asr_ref[...], hw, preferred_element_type=_f32)
    ad2_ref[...] = jnp.dot(adr_ref[...], hw, preferred_element_type=_f32)


def _fold_encode(acc, den, hwt, as_, ad, b, w, asr, adr):
    return pl.pallas_call(
        _fold_enc_body,
        grid=(CGRID,),
        in_specs=[
            pl.BlockSpec((D, CBLK), lambda i: (0, i)),
            pl.BlockSpec((NW, CBLK), lambda i: (0, i)),
            pl.BlockSpec((D, CBLK), lambda i: (0, i)),
            pl.BlockSpec((1, CBLK), lambda i: (0, i)),
            pl.BlockSpec((1, CBLK), lambda i: (0, i)),
            pl.BlockSpec((D, 1), lambda i: (0, 0)),
            pl.BlockSpec((D, D), lambda i: (0, 0)),
            pl.BlockSpec((1, D), lambda i: (0, 0)),
            pl.BlockSpec((1, D), lambda i: (0, 0)),
        ],
        out_specs=[
            pl.BlockSpec((D, CBLK), lambda i: (0, i)),
            pl.BlockSpec((1, CBLK), lambda i: (0, i)),
            pl.BlockSpec((1, CBLK), lambda i: (0, i)),
        ],
        out_shape=[
            jax.ShapeDtypeStruct((D, N), _f32),
            jax.ShapeDtypeStruct((1, N), _f32),
            jax.ShapeDtypeStruct((1, N), _f32),
        ],
    )(acc, den, hwt, as_, ad, b, w, asr, adr)


def _final_body(acc_ref, den_ref, hwt_ref, as_ref, ad_ref, b_ref, out_ref):
    out_ref[...] = _fold(acc_ref, den_ref, hwt_ref, as_ref, ad_ref, b_ref)


def _final(acc, den, hwt, as_, ad, b):
    return pl.pallas_call(
        _final_body,
        grid=(CGRID,),
        in_specs=[
            pl.BlockSpec((D, CBLK), lambda i: (0, i)),
            pl.BlockSpec((NW, CBLK), lambda i: (0, i)),
            pl.BlockSpec((D, CBLK), lambda i: (0, i)),
            pl.BlockSpec((1, CBLK), lambda i: (0, i)),
            pl.BlockSpec((1, CBLK), lambda i: (0, i)),
            pl.BlockSpec((D, 1), lambda i: (0, 0)),
        ],
        out_specs=pl.BlockSpec((D, CBLK), lambda i: (0, i)),
        out_shape=jax.ShapeDtypeStruct((D, N), _f32),
    )(acc, den, hwt, as_, ad, b)


_mesh = plsc.VectorSubcoreMesh(core_axis_name="c", subcore_axis_name="s")


@functools.partial(
    pl.kernel,
    out_type=[
        jax.ShapeDtypeStruct((EPAD,), _f32),
        jax.ShapeDtypeStruct((NW, 1, NDEN), _f32),
    ],
    mesh=_mesh,
    compiler_params=pltpu.CompilerParams(needs_layout_passes=False),
    scratch_types=[
        pltpu.VMEM((N,), _f32),
        pltpu.VMEM((N + L,), _f32),
        pltpu.VMEM((1, NDEN), _f32),
        pltpu.VMEM((2, EPT), jnp.int32),
        pltpu.VMEM((EPT,), _f32),
    ],
)
def _agg_w(as_hbm, ad_hbm, ei_hbm, w_hbm, den_hbm,
           asv, adv, denv, ebuf, wbuf):
    c = lax.axis_index("c")
    s = lax.axis_index("s")
    wid = s * NC + c
    base = wid * EPT

    pltpu.sync_copy(as_hbm, asv)
    pltpu.sync_copy(ad_hbm, adv.at[pl.ds(0, N)])
    pltpu.sync_copy(ei_hbm.at[:, pl.ds(base, EPT)], ebuf)
    adv[pl.ds(N, L)] = jnp.zeros((L,), _f32)

    def zero_body(i, _):
        denv[0, pl.ds(i * L, L)] = jnp.zeros((L,), _f32)
        return 0

    lax.fori_loop(0, NDEN // L, zero_body, 0, unroll=False)

    zrow = jnp.zeros((L,), jnp.int32)

    @plsc.parallel_loop(0, EPT // L, unroll=4)
    def _(j):
        sidx = ebuf[0, pl.ds(j * L, L)]
        didx = ebuf[1, pl.ds(j * L, L)]
        t = plsc.load_gather(asv, [sidx]) + plsc.load_gather(adv, [didx])
        e = jnp.where(t >= 0, t, t * _f32(0.2))
        w = jnp.exp(e)
        wbuf[pl.ds(j * L, L)] = w
        plsc.addupdate_scatter(denv, [zrow, didx], w)

    pltpu.sync_copy(wbuf, w_hbm.at[pl.ds(base, EPT)])
    pltpu.sync_copy(denv, den_hbm.at[wid])


@functools.partial(
    pl.kernel,
    out_type=jax.ShapeDtypeStruct((NW, FPT, N), _f32),
    mesh=_mesh,
    compiler_params=pltpu.CompilerParams(needs_layout_passes=False),
    scratch_types=[
        pltpu.VMEM((FPT, N), _f32),
        pltpu.VMEM((FPT, N), _f32),
        pltpu.VMEM((2, 2, CH), jnp.int32),
        pltpu.VMEM((2, CH), _f32),
        pltpu.SemaphoreType.DMA,
        pltpu.SemaphoreType.DMA,
        pltpu.SemaphoreType.DMA,
        pltpu.SemaphoreType.DMA,
    ],
)
def _agg_acc(hw_hbm, ei_hbm, w_hbm, acc_hbm,
             hwv, accv, ebuf, wvbuf, esem0, esem1, wsem0, wsem1):
    c = lax.axis_index("c")
    s = lax.axis_index("s")
    wid = s * NC + c

    pltpu.sync_copy(hw_hbm.at[wid], hwv)

    def zero_body(i, _):
        z = jnp.zeros((L,), _f32)
        for f in range(FPT):
            accv[f, pl.ds(i * L, L)] = z
        return 0

    lax.fori_loop(0, N // L, zero_body, 0, unroll=False)

    esems = (esem0, esem1)
    wsems = (wsem0, wsem1)
    for b in range(2):
        pltpu.async_copy(ei_hbm.at[:, pl.ds(b * CH, CH)], ebuf.at[b], esems[b])
        pltpu.async_copy(w_hbm.at[pl.ds(b * CH, CH)], wvbuf.at[b], wsems[b])

    def chunk_body(g2, _):
        for b in range(2):
            g = g2 * 2 + b
            pltpu.make_async_copy(
                ei_hbm.at[:, pl.ds(0, CH)], ebuf.at[b], esems[b]).wait()
            pltpu.make_async_copy(
                w_hbm.at[pl.ds(0, CH)], wvbuf.at[b], wsems[b]).wait()

            @plsc.parallel_loop(0, CH // L, unroll=8)
            def _(j):
                sidx = ebuf[b, 0, pl.ds(j * L, L)]
                didx = ebuf[b, 1, pl.ds(j * L, L)]
                w = wvbuf[b, pl.ds(j * L, L)]
                for f in range(FPT):
                    fv = jnp.full((L,), f, jnp.int32)
                    rows = plsc.load_gather(hwv, [fv, sidx])
                    plsc.addupdate_scatter(accv, [fv, didx], rows * w)

            @pl.when(g + 2 < NCHUNK)
            def _():
                pltpu.async_copy(
                    ei_hbm.at[:, pl.ds((g + 2) * CH, CH)], ebuf.at[b], esems[b])
                pltpu.async_copy(
                    w_hbm.at[pl.ds((g + 2) * CH, CH)], wvbuf.at[b], wsems[b])
        return 0

    lax.fori_loop(0, NCHUNK // 2, chunk_body, 0, unroll=False)

    pltpu.sync_copy(accv, acc_hbm.at[wid])


def kernel(x, edge_index, edge_attr, Wn, bn, We, be,
           l0_We, l0_be, l0_W, l0_asrc, l0_adst, l0_bias,
           l1_We, l1_be, l1_W, l1_asrc, l1_adst, l1_bias):
    xt = x.T
    ei = edge_index.astype(jnp.int32)
    pad = jnp.stack([jnp.zeros((EPAD - E,), jnp.int32),
                     jnp.full((EPAD - E,), N, jnp.int32)])
    eip = jnp.concatenate([ei, pad], axis=1)

    hw0, as0, ad0 = _encode(xt, Wn, bn.reshape(D, 1), l0_W,
                            l0_asrc.reshape(1, D), l0_adst.reshape(1, D))
    w0, den0 = _agg_w(as0.reshape(N), ad0.reshape(N), eip)
    acc0 = _agg_acc(hw0.reshape(NW, FPT, N), eip, w0)
    hw1, as1, ad1 = _fold_encode(acc0.reshape(D, N), den0.reshape(NW, NDEN),
                                 hw0, as0, ad0,
                                 l0_bias.reshape(D, 1), l1_W,
                                 l1_asrc.reshape(1, D), l1_adst.reshape(1, D))
    w1, den1 = _agg_w(as1.reshape(N), ad1.reshape(N), eip)
    acc1 = _agg_acc(hw1.reshape(NW, FPT, N), eip, w1)
    outt = _final(acc1.reshape(D, N), den1.reshape(NW, NDEN),
                  hw1, as1, ad1, l1_bias.reshape(D, 1))
    return outt.T


# speedup vs baseline: 1.0291x; 1.0291x over previous
import functools

import jax
import jax.numpy as jnp
from jax import lax
from jax.experimental import pallas as pl
from jax.experimental.pallas import tpu as pltpu
from jax.experimental.pallas import tpu_sc as plsc

N = 10000
E = 320000
D = 128

NC = 2
NS = 16
L = 16
NW = NC * NS
FPT = D // NW
EPT = 10240
EPAD = NW * EPT
NDEN = 10240
CH = 1280
NCHUNK = E // CH

CBLK = 1024
CGRID = -(-N // CBLK)

_f32 = jnp.float32


def _enc_body(xt_ref, wn_ref, bn_ref, w_ref, asr_ref, adr_ref,
              hwt_ref, as_ref, ad_ref):
    h = jnp.dot(wn_ref[...], xt_ref[...], preferred_element_type=_f32) + bn_ref[...]
    hw = jnp.dot(w_ref[...], h, preferred_element_type=_f32)
    hwt_ref[...] = hw
    as_ref[...] = jnp.dot(asr_ref[...], hw, preferred_element_type=_f32)
    ad_ref[...] = jnp.dot(adr_ref[...], hw, preferred_element_type=_f32)


def _encode(xt, wn, bn, w, asr, adr):
    return pl.pallas_call(
        _enc_body,
        grid=(CGRID,),
        in_specs=[
            pl.BlockSpec((D, CBLK), lambda i: (0, i)),
            pl.BlockSpec((D, D), lambda i: (0, 0)),
            pl.BlockSpec((D, 1), lambda i: (0, 0)),
            pl.BlockSpec((D, D), lambda i: (0, 0)),
            pl.BlockSpec((1, D), lambda i: (0, 0)),
            pl.BlockSpec((1, D), lambda i: (0, 0)),
        ],
        out_specs=[
            pl.BlockSpec((D, CBLK), lambda i: (0, i)),
            pl.BlockSpec((1, CBLK), lambda i: (0, i)),
            pl.BlockSpec((1, CBLK), lambda i: (0, i)),
        ],
        out_shape=[
            jax.ShapeDtypeStruct((D, N), _f32),
            jax.ShapeDtypeStruct((1, N), _f32),
            jax.ShapeDtypeStruct((1, N), _f32),
        ],
    )(xt, wn, bn, w, asr, adr)


def _fold(acc_ref, den_ref, hwt_ref, as_ref, ad_ref, b_ref):
    t = as_ref[...] + ad_ref[...]
    e = jnp.where(t >= 0, t, t * _f32(0.2))
    ws = jnp.exp(e)
    den = jnp.sum(den_ref[...], axis=0, keepdims=True)
    return (acc_ref[...] + ws * hwt_ref[...]) / (den + ws) + b_ref[...]


def _fold_enc_body(acc_ref, den_ref, hwt_ref, as_ref, ad_ref, b_ref,
                   w_ref, asr_ref, adr_ref, hwt2_ref, as2_ref, ad2_ref):
    h = _fold(acc_ref, den_ref, hwt_ref, as_ref, ad_ref, b_ref)
    hw = jnp.dot(w_ref[...], h, preferred_element_type=_f32)
    hwt2_ref[...] = hw
    as2_ref[...] = jnp.dot(asr_ref[...], hw, preferred_element_type=_f32)
    ad2_ref[...] = jnp.dot(adr_ref[...], hw, preferred_element_type=_f32)


def _fold_encode(acc, den, hwt, as_, ad, b, w, asr, adr):
    return pl.pallas_call(
        _fold_enc_body,
        grid=(CGRID,),
        in_specs=[
            pl.BlockSpec((D, CBLK), lambda i: (0, i)),
            pl.BlockSpec((NW, CBLK), lambda i: (0, i)),
            pl.BlockSpec((D, CBLK), lambda i: (0, i)),
            pl.BlockSpec((1, CBLK), lambda i: (0, i)),
            pl.BlockSpec((1, CBLK), lambda i: (0, i)),
            pl.BlockSpec((D, 1), lambda i: (0, 0)),
            pl.BlockSpec((D, D), lambda i: (0, 0)),
            pl.BlockSpec((1, D), lambda i: (0, 0)),
            pl.BlockSpec((1, D), lambda i: (0, 0)),
        ],
        out_specs=[
            pl.BlockSpec((D, CBLK), lambda i: (0, i)),
            pl.BlockSpec((1, CBLK), lambda i: (0, i)),
            pl.BlockSpec((1, CBLK), lambda i: (0, i)),
        ],
        out_shape=[
            jax.ShapeDtypeStruct((D, N), _f32),
            jax.ShapeDtypeStruct((1, N), _f32),
            jax.ShapeDtypeStruct((1, N), _f32),
        ],
    )(acc, den, hwt, as_, ad, b, w, asr, adr)


def _final_body(acc_ref, den_ref, hwt_ref, as_ref, ad_ref, b_ref, out_ref):
    out_ref[...] = _fold(acc_ref, den_ref, hwt_ref, as_ref, ad_ref, b_ref)


def _final(acc, den, hwt, as_, ad, b):
    return pl.pallas_call(
        _final_body,
        grid=(CGRID,),
        in_specs=[
            pl.BlockSpec((D, CBLK), lambda i: (0, i)),
            pl.BlockSpec((NW, CBLK), lambda i: (0, i)),
            pl.BlockSpec((D, CBLK), lambda i: (0, i)),
            pl.BlockSpec((1, CBLK), lambda i: (0, i)),
            pl.BlockSpec((1, CBLK), lambda i: (0, i)),
            pl.BlockSpec((D, 1), lambda i: (0, 0)),
        ],
        out_specs=pl.BlockSpec((D, CBLK), lambda i: (0, i)),
        out_shape=jax.ShapeDtypeStruct((D, N), _f32),
    )(acc, den, hwt, as_, ad, b)


_mesh = plsc.VectorSubcoreMesh(core_axis_name="c", subcore_axis_name="s")


@functools.partial(
    pl.kernel,
    out_type=[
        jax.ShapeDtypeStruct((EPAD,), _f32),
        jax.ShapeDtypeStruct((NW, 1, NDEN), _f32),
        jax.ShapeDtypeStruct((EPAD,), jnp.int32),
    ],
    mesh=_mesh,
    compiler_params=pltpu.CompilerParams(needs_layout_passes=False),
    scratch_types=[
        pltpu.VMEM((N,), _f32),
        pltpu.VMEM((N + L,), _f32),
        pltpu.VMEM((1, NDEN), _f32),
        pltpu.VMEM((2, EPT), jnp.int32),
        pltpu.VMEM((EPT,), _f32),
        pltpu.VMEM((EPT,), jnp.int32),
    ],
)
def _agg_w(as_hbm, ad_hbm, ei_hbm, w_hbm, den_hbm, pk_hbm,
           asv, adv, denv, ebuf, wbuf, pbuf):
    c = lax.axis_index("c")
    s = lax.axis_index("s")
    wid = s * NC + c
    base = wid * EPT

    pltpu.sync_copy(as_hbm, asv)
    pltpu.sync_copy(ad_hbm, adv.at[pl.ds(0, N)])
    pltpu.sync_copy(ei_hbm.at[:, pl.ds(base, EPT)], ebuf)
    adv[pl.ds(N, L)] = jnp.zeros((L,), _f32)

    def zero_body(i, _):
        denv[0, pl.ds(i * L, L)] = jnp.zeros((L,), _f32)
        return 0

    lax.fori_loop(0, NDEN // L, zero_body, 0, unroll=False)

    zrow = jnp.zeros((L,), jnp.int32)

    @plsc.parallel_loop(0, EPT // L, unroll=8)
    def _(j):
        sidx = ebuf[0, pl.ds(j * L, L)]
        didx = ebuf[1, pl.ds(j * L, L)]
        t = plsc.load_gather(asv, [sidx]) + plsc.load_gather(adv, [didx])
        e = jnp.where(t >= 0, t, t * _f32(0.2))
        w = jnp.exp(e)
        wbuf[pl.ds(j * L, L)] = w
        pbuf[pl.ds(j * L, L)] = jnp.bitwise_or(
            jnp.left_shift(sidx, 14), didx)
        plsc.addupdate_scatter(denv, [zrow, didx], w)

    pltpu.sync_copy(wbuf, w_hbm.at[pl.ds(base, EPT)])
    pltpu.sync_copy(pbuf, pk_hbm.at[pl.ds(base, EPT)])
    pltpu.sync_copy(denv, den_hbm.at[wid])


@functools.partial(
    pl.kernel,
    out_type=jax.ShapeDtypeStruct((NW, FPT, N), _f32),
    mesh=_mesh,
    compiler_params=pltpu.CompilerParams(needs_layout_passes=False),
    scratch_types=[
        pltpu.VMEM((FPT, N), _f32),
        pltpu.VMEM((FPT, N), _f32),
        pltpu.VMEM((2, CH), jnp.int32),
        pltpu.VMEM((2, CH), _f32),
        pltpu.SemaphoreType.DMA,
        pltpu.SemaphoreType.DMA,
        pltpu.SemaphoreType.DMA,
        pltpu.SemaphoreType.DMA,
    ],
)
def _agg_acc(hw_hbm, pk_hbm, w_hbm, acc_hbm,
             hwv, accv, ebuf, wvbuf, esem0, esem1, wsem0, wsem1):
    c = lax.axis_index("c")
    s = lax.axis_index("s")
    wid = s * NC + c

    pltpu.sync_copy(hw_hbm.at[wid], hwv)

    def zero_body(i, _):
        z = jnp.zeros((L,), _f32)
        for f in range(FPT):
            accv[f, pl.ds(i * L, L)] = z
        return 0

    lax.fori_loop(0, N // L, zero_body, 0, unroll=False)

    esems = (esem0, esem1)
    wsems = (wsem0, wsem1)
    for b in range(2):
        pltpu.async_copy(pk_hbm.at[pl.ds(b * CH, CH)], ebuf.at[b], esems[b])
        pltpu.async_copy(w_hbm.at[pl.ds(b * CH, CH)], wvbuf.at[b], wsems[b])

    def chunk_body(g2, _):
        for b in range(2):
            g = g2 * 2 + b
            pltpu.make_async_copy(
                pk_hbm.at[pl.ds(0, CH)], ebuf.at[b], esems[b]).wait()
            pltpu.make_async_copy(
                w_hbm.at[pl.ds(0, CH)], wvbuf.at[b], wsems[b]).wait()

            @plsc.parallel_loop(0, CH // L, unroll=8)
            def _(j):
                p = ebuf[b, pl.ds(j * L, L)]
                sidx = jnp.right_shift(p, 14)
                didx = jnp.bitwise_and(p, 16383)
                w = wvbuf[b, pl.ds(j * L, L)]
                for f in range(FPT):
                    fv = jnp.full((L,), f, jnp.int32)
                    rows = plsc.load_gather(hwv, [fv, sidx])
                    plsc.addupdate_scatter(accv, [fv, didx], rows * w)

            @pl.when(g + 2 < NCHUNK)
            def _():
                pltpu.async_copy(
                    pk_hbm.at[pl.ds((g + 2) * CH, CH)], ebuf.at[b], esems[b])
                pltpu.async_copy(
                    w_hbm.at[pl.ds((g + 2) * CH, CH)], wvbuf.at[b], wsems[b])
        return 0

    lax.fori_loop(0, NCHUNK // 2, chunk_body, 0, unroll=False)

    pltpu.sync_copy(accv, acc_hbm.at[wid])


def kernel(x, edge_index, edge_attr, Wn, bn, We, be,
           l0_We, l0_be, l0_W, l0_asrc, l0_adst, l0_bias,
           l1_We, l1_be, l1_W, l1_asrc, l1_adst, l1_bias):
    xt = x.T
    ei = edge_index.astype(jnp.int32)
    pad = jnp.stack([jnp.zeros((EPAD - E,), jnp.int32),
                     jnp.full((EPAD - E,), N, jnp.int32)])
    eip = jnp.concatenate([ei, pad], axis=1)

    hw0, as0, ad0 = _encode(xt, Wn, bn.reshape(D, 1), l0_W,
                            l0_asrc.reshape(1, D), l0_adst.reshape(1, D))
    w0, den0, pk0 = _agg_w(as0.reshape(N), ad0.reshape(N), eip)
    acc0 = _agg_acc(hw0.reshape(NW, FPT, N), pk0, w0)
    hw1, as1, ad1 = _fold_encode(acc0.reshape(D, N), den0.reshape(NW, NDEN),
                                 hw0, as0, ad0,
                                 l0_bias.reshape(D, 1), l1_W,
                                 l1_asrc.reshape(1, D), l1_adst.reshape(1, D))
    w1, den1, _pk1 = _agg_w(as1.reshape(N), ad1.reshape(N), eip)
    acc1 = _agg_acc(hw1.reshape(NW, FPT, N), pk0, w1)
    outt = _final(acc1.reshape(D, N), den1.reshape(NW, NDEN),
                  hw1, as1, ad1, l1_bias.reshape(D, 1))
    return outt.T
